# baseline (speedup 1.0000x reference)
import functools

import jax
import jax.numpy as jnp
from jax import lax
from jax.experimental import pallas as pl
from jax.experimental.pallas import tpu as pltpu
from jax.experimental.pallas import tpu_sc as plsc

D_MODEL = 64
L = 128
R = 8
_INFO = plsc.get_sparse_core_info()
NC = _INFO.num_cores
NS = _INFO.num_subcores
NW = NC * NS


def _embed_body(tok_hbm, tab_hbm, out_hbm, idx_v, rows_v, sem):
    nrows_total = tok_hbm.shape[0]
    nr = nrows_total // NW
    wid = lax.axis_index("s") * NC + lax.axis_index("c")
    row0 = wid * nr

    def chunk(g, _):
        r0 = row0 + g * R
        pltpu.sync_copy(tok_hbm.at[pl.ds(r0, R)], idx_v)
        copies = []
        for j in range(R):
            copies.append(
                pltpu.async_copy(tab_hbm.at[idx_v.at[j]],
                                 rows_v.at[pl.ds(j * L, L)], sem))
        for c in copies:
            c.wait()
        pltpu.sync_copy(rows_v, out_hbm.at[pl.ds(r0 * L, R * L)])
        return 0

    lax.fori_loop(0, nr // R, chunk, 0)


@jax.jit
def kernel(tokens, W_E):
    B, S = tokens.shape
    total = B * S
    tok2d = tokens.reshape(total // L, L)
    mesh = plsc.VectorSubcoreMesh(core_axis_name="c", subcore_axis_name="s")
    out = pl.kernel(
        _embed_body,
        mesh=mesh,
        out_type=jax.ShapeDtypeStruct((total, D_MODEL), jnp.float32),
        scratch_types=[
            pltpu.VMEM((R, L), jnp.int32),
            pltpu.VMEM((R * L, D_MODEL), jnp.float32),
            pltpu.SemaphoreType.DMA,
        ],
        compiler_params=pltpu.CompilerParams(use_tc_tiling_on_sc=False),
    )(tok2d, W_E)
    return out.reshape(B, S, D_MODEL)


# speedup vs baseline: 1.0149x; 1.0149x over previous
import functools

import jax
import jax.numpy as jnp
from jax import lax
from jax.experimental import pallas as pl
from jax.experimental.pallas import tpu as pltpu
from jax.experimental.pallas import tpu_sc as plsc

D_MODEL = 64
L = 128
R = 4
_INFO = plsc.get_sparse_core_info()
NC = _INFO.num_cores
NS = _INFO.num_subcores
NW = NC * NS


def _embed_body(tok_hbm, tab_hbm, out_hbm,
                idx0, idx1, rows0, rows1,
                isem0, isem1, gsem0, gsem1, ssem0, ssem1):
    nrows_total = tok_hbm.shape[0]
    nr = nrows_total // NW
    ch = nr // R
    wid = lax.axis_index("s") * NC + lax.axis_index("c")
    row0 = wid * nr

    bufs = ((idx0, rows0, isem0, gsem0, ssem0),
            (idx1, rows1, isem1, gsem1, ssem1))

    def do_chunk(r0, b, wait_store, idx_prefetch_r0):
        idx_b, rows_b, isem_b, gsem_b, ssem_b = bufs[b]
        if wait_store:
            pltpu.make_async_copy(
                rows_b, out_hbm.at[pl.ds(0, R * L)], ssem_b).wait()
        pltpu.make_async_copy(
            tok_hbm.at[pl.ds(0, R)], idx_b, isem_b).wait()
        gathers = [
            pltpu.async_copy(tab_hbm.at[idx_b.at[j]],
                             rows_b.at[pl.ds(j * L, L)], gsem_b)
            for j in range(R)
        ]
        for c in gathers:
            c.wait()
        if idx_prefetch_r0 is not None:
            pltpu.async_copy(
                tok_hbm.at[pl.ds(idx_prefetch_r0, R)], idx_b, isem_b)
        pltpu.async_copy(rows_b, out_hbm.at[pl.ds(r0 * L, R * L)], ssem_b)

    pltpu.async_copy(tok_hbm.at[pl.ds(row0, R)], idx0, isem0)
    pltpu.async_copy(tok_hbm.at[pl.ds(row0 + R, R)], idx1, isem1)
    do_chunk(row0, 0, False, row0 + 2 * R)
    do_chunk(row0 + R, 1, False, row0 + 3 * R)

    def pair(o, _):
        r0 = row0 + (2 * o) * R
        do_chunk(r0, 0, True, r0 + 2 * R)
        do_chunk(r0 + R, 1, True, r0 + 3 * R)
        return 0

    lax.fori_loop(1, ch // 2 - 1, pair, 0)

    r_last = row0 + (ch - 2) * R
    do_chunk(r_last, 0, True, None)
    do_chunk(r_last + R, 1, True, None)
    pltpu.make_async_copy(rows0, out_hbm.at[pl.ds(0, R * L)], ssem0).wait()
    pltpu.make_async_copy(rows1, out_hbm.at[pl.ds(0, R * L)], ssem1).wait()


@jax.jit
def kernel(tokens, W_E):
    B, S = tokens.shape
    total = B * S
    tok2d = tokens.reshape(total // L, L)
    mesh = plsc.VectorSubcoreMesh(core_axis_name="c", subcore_axis_name="s")
    out = pl.kernel(
        _embed_body,
        mesh=mesh,
        out_type=jax.ShapeDtypeStruct((total, D_MODEL), jnp.float32),
        scratch_types=[
            pltpu.VMEM((R, L), jnp.int32),
            pltpu.VMEM((R, L), jnp.int32),
            pltpu.VMEM((R * L, D_MODEL), jnp.float32),
            pltpu.VMEM((R * L, D_MODEL), jnp.float32),
            pltpu.SemaphoreType.DMA,
            pltpu.SemaphoreType.DMA,
            pltpu.SemaphoreType.DMA,
            pltpu.SemaphoreType.DMA,
            pltpu.SemaphoreType.DMA,
            pltpu.SemaphoreType.DMA,
        ],
        compiler_params=pltpu.CompilerParams(use_tc_tiling_on_sc=False),
    )(tok2d, W_E)
    return out.reshape(B, S, D_MODEL)


# speedup vs baseline: 1.0341x; 1.0190x over previous
import functools

import jax
import jax.numpy as jnp
from jax import lax
from jax.experimental import pallas as pl
from jax.experimental.pallas import tpu as pltpu
from jax.experimental.pallas import tpu_sc as plsc

D_MODEL = 64
S_LEN = 200
_INFO = plsc.get_sparse_core_info()
NC = _INFO.num_cores
NS = _INFO.num_subcores
NW = NC * NS


def _embed_body(tok_hbm, tab_hbm, out_hbm,
                idx0, idx1, rows0, rows1,
                isem0, isem1, gsem0, gsem1, ssem0, ssem1):
    nb = tok_hbm.shape[0] // NW
    wid = lax.axis_index("s") * NC + lax.axis_index("c")
    b0 = wid * nb

    bufs = ((idx0, rows0, isem0, gsem0, ssem0),
            (idx1, rows1, isem1, gsem1, ssem1))

    def do_row(b, buf, wait_store, prefetch_b):
        idx_b, rows_b, isem_b, gsem_b, ssem_b = bufs[buf]
        if wait_store:
            pltpu.make_async_copy(rows_b, out_hbm.at[0], ssem_b).wait()
        pltpu.make_async_copy(tok_hbm.at[0], idx_b, isem_b).wait()
        c1 = pltpu.async_copy(tab_hbm.at[idx_b.at[pl.ds(0, 128)]],
                              rows_b.at[pl.ds(0, 128)], gsem_b)
        c2 = pltpu.async_copy(tab_hbm.at[idx_b.at[pl.ds(128, S_LEN - 128)]],
                              rows_b.at[pl.ds(128, S_LEN - 128)], gsem_b)
        c1.wait()
        c2.wait()
        if prefetch_b is not None:
            pltpu.async_copy(tok_hbm.at[prefetch_b], idx_b, isem_b)
        pltpu.async_copy(rows_b, out_hbm.at[b], ssem_b)

    pltpu.async_copy(tok_hbm.at[b0], idx0, isem0)
    pltpu.async_copy(tok_hbm.at[b0 + 1], idx1, isem1)
    do_row(b0, 0, False, b0 + 2)
    do_row(b0 + 1, 1, False, b0 + 3)

    def pair(o, _):
        b = b0 + 2 * o
        do_row(b, 0, True, b + 2)
        do_row(b + 1, 1, True, b + 3)
        return 0

    lax.fori_loop(1, nb // 2 - 1, pair, 0)

    b_last = b0 + nb - 2
    do_row(b_last, 0, True, None)
    do_row(b_last + 1, 1, True, None)
    pltpu.make_async_copy(rows0, out_hbm.at[0], ssem0).wait()
    pltpu.make_async_copy(rows1, out_hbm.at[0], ssem1).wait()


@jax.jit
def kernel(tokens, W_E):
    B, S = tokens.shape
    V, D = W_E.shape
    W2 = jnp.pad(W_E, ((0, 0), (0, 128 - D))).reshape(2 * V, D)
    tok2 = tokens * 2
    mesh = plsc.VectorSubcoreMesh(core_axis_name="c", subcore_axis_name="s")
    return pl.kernel(
        _embed_body,
        mesh=mesh,
        out_type=jax.ShapeDtypeStruct((B, S, D_MODEL), jnp.float32),
        scratch_types=[
            pltpu.VMEM((S_LEN,), jnp.int32),
            pltpu.VMEM((S_LEN,), jnp.int32),
            pltpu.VMEM((S_LEN, D_MODEL), jnp.float32),
            pltpu.VMEM((S_LEN, D_MODEL), jnp.float32),
            pltpu.SemaphoreType.DMA,
            pltpu.SemaphoreType.DMA,
            pltpu.SemaphoreType.DMA,
            pltpu.SemaphoreType.DMA,
            pltpu.SemaphoreType.DMA,
            pltpu.SemaphoreType.DMA,
        ],
        compiler_params=pltpu.CompilerParams(use_tc_tiling_on_sc=False),
    )(tok2, W2)
